# Initial kernel scaffold; baseline (speedup 1.0000x reference)
import jax
import jax.numpy as jnp
from jax.experimental import pallas as pl


def kernel(x, rots, trans, intrins, W_depth, b_depth):
    raise NotImplementedError("write your pallas kernel here")


# speedup vs baseline: 1.0312x; 1.0312x over previous
import jax
import jax.numpy as jnp
from jax import lax
from jax.experimental import pallas as pl

_B, _N, _CAM_C, _OUT_C = 2, 6, 512, 80
_FH, _FW = 16, 44
_P = _FH * _FW
_D = 59
_DP = 64
_NG = 128
_VOX = _NG * _NG
_BN = _B * _N
_PIX = _BN * _P


def _tc_body(xt_ref, wd_ref, wc_ref, bd_ref, bc_ref, geo_ref, uv_ref, dv_ref,
             depth_ref, ctx_ref, rank_ref):
    xt = xt_ref[0]
    yd = lax.dot_general(xt, wd_ref[...], (((1,), (0,)), ((), ())),
                         preferred_element_type=jnp.float32) + bd_ref[...]
    yc = lax.dot_general(xt, wc_ref[...], (((1,), (0,)), ((), ())),
                         preferred_element_type=jnp.float32) + bc_ref[...]
    m = jnp.max(yd, axis=1, keepdims=True)
    e = jnp.exp(yd - m)
    depth_ref[0] = e / jnp.sum(e, axis=1, keepdims=True)
    ctx_ref[0] = yc
    u = uv_ref[:, 0:1]
    v = uv_ref[:, 1:2]
    dv = dv_ref[...]
    g = geo_ref[...]

    def _bf(z):
        return z.astype(jnp.bfloat16).astype(jnp.float32)

    ud = _bf(u * dv)
    vd = _bf(v * dv)
    dvb = _bf(jnp.broadcast_to(dv, ud.shape))

    def _coord(r):
        return (_bf(g[0, 0, 3 * r]) * ud + _bf(g[0, 0, 3 * r + 1]) * vd
                + _bf(g[0, 0, 3 * r + 2]) * dvb) + g[0, 0, 9 + r]

    gx = ((_coord(0) - (-51.2)) / 0.8).astype(jnp.int32)
    gy = ((_coord(1) - (-51.2)) / 0.8).astype(jnp.int32)
    gz = ((_coord(2) - (-5.0)) / 8.0).astype(jnp.int32)
    kept = ((gx >= 0) & (gx < _NG) & (gy >= 0) & (gy < _NG)
            & (gz >= 0) & (gz < 1))
    kept &= lax.broadcasted_iota(jnp.int32, (_P, _DP), 1) < _D
    rank_ref[0] = jnp.where(kept, gx + gy * _NG, _VOX)


def _tc_call(xt, wd, wc, bd, bc, geo, uv, dv):
    return pl.pallas_call(
        _tc_body,
        grid=(_BN,),
        in_specs=[
            pl.BlockSpec((1, _P, _CAM_C), lambda i: (i, 0, 0)),
            pl.BlockSpec((_CAM_C, _DP), lambda i: (0, 0)),
            pl.BlockSpec((_CAM_C, _OUT_C), lambda i: (0, 0)),
            pl.BlockSpec((1, _DP), lambda i: (0, 0)),
            pl.BlockSpec((1, _OUT_C), lambda i: (0, 0)),
            pl.BlockSpec((1, 1, 16), lambda i: (i, 0, 0)),
            pl.BlockSpec((_P, 2), lambda i: (0, 0)),
            pl.BlockSpec((1, _DP), lambda i: (0, 0)),
        ],
        out_specs=[
            pl.BlockSpec((1, _P, _DP), lambda i: (i, 0, 0)),
            pl.BlockSpec((1, _P, _OUT_C), lambda i: (i, 0, 0)),
            pl.BlockSpec((1, _P, _DP), lambda i: (i, 0, 0)),
        ],
        out_shape=[
            jax.ShapeDtypeStruct((_BN, _P, _DP), jnp.float32),
            jax.ShapeDtypeStruct((_BN, _P, _OUT_C), jnp.float32),
            jax.ShapeDtypeStruct((_BN, _P, _DP), jnp.int32),
        ],
    )(xt, wd, wc, bd, bc, geo, uv, dv)


def kernel(x, rots, trans, intrins, W_depth, b_depth):
    xt = x.reshape(_BN, _CAM_C, _P).transpose(0, 2, 1).astype(jnp.bfloat16)
    wd = (jnp.zeros((_CAM_C, _DP), jnp.float32).at[:, :_D]
          .set(W_depth[:_D].T).astype(jnp.bfloat16))
    wc = W_depth[_D:].T.astype(jnp.bfloat16)
    bd = jnp.full((1, _DP), -1e30, jnp.float32).at[0, :_D].set(b_depth[:_D])
    bc = b_depth[_D:].reshape(1, _OUT_C)
    comb = jnp.matmul(rots, jnp.linalg.inv(intrins)).reshape(_BN, 9)
    geo = jnp.concatenate(
        [comb, trans.reshape(_BN, 3), jnp.zeros((_BN, 4), jnp.float32)],
        1).reshape(_BN, 1, 16)
    xs = jnp.linspace(0.0, 704.0 - 1.0, _FW, dtype=jnp.float32)
    ys = jnp.linspace(0.0, 256.0 - 1.0, _FH, dtype=jnp.float32)
    u = jnp.broadcast_to(xs[None, :], (_FH, _FW)).reshape(_P)
    v = jnp.broadcast_to(ys[:, None], (_FH, _FW)).reshape(_P)
    uv = jnp.stack([u, v], axis=1)
    dv = jnp.concatenate(
        [jnp.arange(1.0, 60.0, 1.0, dtype=jnp.float32),
         jnp.ones((_DP - _D,), jnp.float32)]).reshape(1, _DP)

    depth, ctx, rank = _tc_call(xt, wd, wc, bd, bc, geo, uv, dv)
    depth = depth[:, :, :_D].reshape(_B, _N * _P, _D)
    rank = rank[:, :, :_D].reshape(_B, _N * _P, _D)
    ctx = ctx.reshape(_B, _N * _P, _OUT_C)
    boff = jnp.arange(_B, dtype=jnp.int32).reshape(_B, 1, 1) * _VOX
    grank = jnp.where(rank < _VOX, rank + boff, _B * _VOX)
    feats = depth[:, :, :, None] * ctx[:, :, None, :]
    bev = (jnp.zeros((_B * _VOX + 1, _OUT_C), jnp.float32)
           .at[grank.reshape(-1)].add(feats.reshape(-1, _OUT_C))[:_B * _VOX])
    return bev.reshape(_B, _NG, _NG, _OUT_C).transpose(0, 3, 1, 2)
